# baseline (speedup 1.0000x reference)
import jax
import jax.numpy as jnp
from jax import lax
from jax.experimental import pallas as pl
from jax.experimental.pallas import tpu as pltpu
from jax.experimental.pallas import tpu_sc as plsc

N1 = 50000
NP = 50048
NG = 64
EPS = 0.1
ORDER = 5
DQ = 16
NSUB = 16
EB = 128
CHUNK = 8
CE = CHUNK * EB
RSLICE = NP // NSUB
_SC_PARAMS = dict(compiler_params=pltpu.CompilerParams(use_tc_tiling_on_sc=False))
_MESH = dict(mesh=plsc.VectorSubcoreMesh(core_axis_name="c", subcore_axis_name="s"))


def _wv16(buf, g):
    j = g // (EB // 16)
    m = pl.multiple_of((g % (EB // 16)) * 16, 16)
    return buf[j, pl.ds(m, 16)]


def _zero_rows(ref, n):
    def zb(i, cc):
        ref[i, :] = jnp.zeros((16,), jnp.float32)
        return cc
    lax.fori_loop(0, n, zb, 0, unroll=8)


_BLOCKS = ((0, CHUNK * EB), (1, CHUNK * EB), (2, RSLICE - 2 * CHUNK * EB))


def _init_accum_zero(zsrc, accum, off):
    for t, sz in _BLOCKS:
        pltpu.sync_copy(zsrc.at[pl.ds(0, sz)],
                        accum.at[pl.ds(off + t * CE, sz)])


def _spmm_builder(nrows, scaled):
    rows_per_tile = nrows // NSUB
    n_chunks = rows_per_tile // CHUNK
    assert n_chunks % 2 == 0

    def body(src, i2r, wr, dinv, base, out,
             ibA, ibB, wbA, wbB, gbA, gbB, accum, gsA, gsB, ssA, ssB, fsA, fsB):
        c = lax.axis_index("c")
        s = lax.axis_index("s")
        row0 = s * rows_per_tile
        off = s * RSLICE

        def fdesc(k, ib, wb, fs):
            r0 = row0 + k * CHUNK
            return [pltpu.make_async_copy(i2r.at[pl.ds(r0, CHUNK)], ib, fs),
                    pltpu.make_async_copy(wr.at[pl.ds(r0 * EB, CE)], wb, fs)]

        def escale(gb, wb):
            def eb(e, cc):
                gb[e, :] = gb[e, :] * wb[e, :]
                return cc
            lax.fori_loop(0, CE, eb, 0, unroll=8)

        for qi in range(2):
            q = 2 * c + qi

            def gdesc(ib, gb):
                return [pltpu.make_async_copy(src.at[q].at[ib.at[j, 1]],
                                              gb.at[pl.ds(j * EB, EB)],
                                              gsA if gb is gbA else gsB)
                        for j in range(CHUNK)]

            def sdesc(ib, gb):
                return [pltpu.make_async_copy(gb.at[pl.ds(j * EB, EB)],
                                              accum.at[ib.at[j, 0]],
                                              ssA if gb is gbA else ssB)
                        for j in range(CHUNK)]

            if scaled:
                _zero_rows(gbA, CE)
                _init_accum_zero(gbA, accum, off)
            else:
                pltpu.sync_copy(base.at[q].at[pl.ds(off, RSLICE)],
                                accum.at[pl.ds(off, RSLICE)])
            plsc.subcore_barrier()

            for d in fdesc(0, ibA, wbA, fsA):
                d.start()
            for d in fdesc(0, ibA, wbA, fsA):
                d.wait()
            for d in gdesc(ibA, gbA):
                d.start()

            def half(k, ib, wb, gb, fs, ibo, wbo, gbo, fso):
                @pl.when(k >= 1)
                def _():
                    for d in sdesc(ibo, gbo):
                        d.wait()

                @pl.when(k + 1 < n_chunks)
                def _():
                    for d in fdesc(k + 1, ibo, wbo, fso):
                        d.start()

                for d in gdesc(ib, gb):
                    d.wait()
                escale(gb, wb)

                @pl.when(k + 1 < n_chunks)
                def _():
                    for d in fdesc(k + 1, ibo, wbo, fso):
                        d.wait()
                    for d in gdesc(ibo, gbo):
                        d.start()

                for d in sdesc(ib, gb):
                    d.start(add=True)

            def pair(t, carry):
                k0 = 2 * t
                half(k0, ibA, wbA, gbA, fsA, ibB, wbB, gbB, fsB)
                half(k0 + 1, ibB, wbB, gbB, fsB, ibA, wbA, gbA, fsA)
                return carry

            lax.fori_loop(0, n_chunks // 2, pair, 0)
            for d in sdesc(ibB, gbB):
                d.wait()
            plsc.subcore_barrier()
            if scaled:
                for t, sz in _BLOCKS:
                    rb = off + t * CE
                    pltpu.sync_copy(accum.at[pl.ds(rb, sz)], gbA.at[pl.ds(0, sz)])
                    pltpu.sync_copy(base.at[q].at[pl.ds(rb, sz)], gbB.at[pl.ds(0, sz)])
                    pltpu.sync_copy(dinv.at[pl.ds(rb, sz)], wbA.at[pl.ds(0, sz)])

                    def wfix(r, cc):
                        gbA[r, :] = gbA[r, :] * wbA[r, :] + gbB[r, :]
                        return cc

                    lax.fori_loop(0, sz, wfix, 0, unroll=8)
                    pltpu.sync_copy(gbA.at[pl.ds(0, sz)],
                                    out.at[q].at[pl.ds(rb, sz)])
            else:
                pltpu.sync_copy(accum.at[pl.ds(off, RSLICE)],
                                out.at[q].at[pl.ds(off, RSLICE)])
            plsc.subcore_barrier()

    scratch = [
        pltpu.VMEM((CHUNK, 2, EB), jnp.int32),
        pltpu.VMEM((CHUNK, 2, EB), jnp.int32),
        pltpu.VMEM((CE, DQ), jnp.float32),
        pltpu.VMEM((CE, DQ), jnp.float32),
        pltpu.VMEM((CE, DQ), jnp.float32),
        pltpu.VMEM((CE, DQ), jnp.float32),
        pltpu.VMEM_SHARED((NP, DQ), jnp.float32),
        pltpu.SemaphoreType.DMA,
        pltpu.SemaphoreType.DMA,
        pltpu.SemaphoreType.DMA,
        pltpu.SemaphoreType.DMA,
        pltpu.SemaphoreType.DMA,
        pltpu.SemaphoreType.DMA,
    ]
    out_t = jax.ShapeDtypeStruct((4, NP, DQ), jnp.float32)
    if scaled:
        def wrapped(src, i2r, wr, dinv, base):
            return pl.kernel(body, out_type=out_t, scratch_types=scratch,
                             **_MESH, **_SC_PARAMS)(src, i2r, wr, dinv, base)
    else:
        def wrapped(src, i2r, wr, base):
            def body2(src, i2r, wr, base, out, *rest):
                return body(src, i2r, wr, None, base, out, *rest)
            return pl.kernel(body2, out_type=out_t, scratch_types=scratch,
                             **_MESH, **_SC_PARAMS)(src, i2r, wr, base)
    return wrapped


def _make_attn(nrows):
    rows_per_core = nrows // 2
    rows_per_tile = rows_per_core // NSUB
    n_chunks = rows_per_tile // CHUNK

    def body(s0x, s1x, i2r, valr, wout, ssout,
             ibuf, vbuf, g0buf, g1buf, ssacc, sem):
        c = lax.axis_index("c")
        s = lax.axis_index("s")
        off = s * RSLICE
        _zero_rows(g0buf, CE)
        _init_accum_zero(g0buf, ssacc, off)
        plsc.subcore_barrier()
        row0 = c * rows_per_core + s * rows_per_tile

        def chunk_body(k, carry):
            r0 = row0 + k * CHUNK
            pltpu.sync_copy(i2r.at[pl.ds(r0, CHUNK)], ibuf)
            pltpu.sync_copy(valr.at[pl.ds(r0, CHUNK)], vbuf)
            cps = [pltpu.async_copy(s0x.at[ibuf.at[j, 0]],
                                    g0buf.at[pl.ds(j * EB, EB)], sem)
                   for j in range(CHUNK)]
            cps += [pltpu.async_copy(s1x.at[ibuf.at[j, 1]],
                                     g1buf.at[pl.ds(j * EB, EB)], sem)
                    for j in range(CHUNK)]
            for cp in cps:
                cp.wait()

            def egrp(g, carry2):
                vv = _wv16(vbuf, g)
                e0 = g * 16
                for l in range(16):
                    e = e0 + l
                    er = g0buf[e, :] + g1buf[e, :]
                    exr = jnp.exp(jnp.maximum(er, 0.2 * er))
                    g0buf[e, :] = exr
                    g1buf[e, :] = exr * vv[l]
                return carry2

            lax.fori_loop(0, CE // 16, egrp, 0)
            for j in range(CHUNK):
                pltpu.sync_copy(g0buf.at[pl.ds(j * EB, EB)],
                                ssacc.at[ibuf.at[j, 0]], add=True)
            pltpu.sync_copy(g1buf, wout.at[pl.ds(r0 * EB, CE)])
            return carry

        lax.fori_loop(0, n_chunks, chunk_body, 0)
        plsc.subcore_barrier()
        pltpu.sync_copy(ssacc.at[pl.ds(off, RSLICE)],
                        ssout.at[c].at[pl.ds(off, RSLICE)])

    def wrapped(s0x, s1x, i2r, valr):
        return pl.kernel(
            body,
            out_type=(jax.ShapeDtypeStruct((nrows * EB, DQ), jnp.float32),
                      jax.ShapeDtypeStruct((2, NP, DQ), jnp.float32)),
            scratch_types=[
                pltpu.VMEM((CHUNK, 2, EB), jnp.int32),
                pltpu.VMEM((CHUNK, EB), jnp.float32),
                pltpu.VMEM((CE, DQ), jnp.float32),
                pltpu.VMEM((CE, DQ), jnp.float32),
                pltpu.VMEM_SHARED((NP, DQ), jnp.float32),
                pltpu.SemaphoreType.DMA,
            ], **_MESH, **_SC_PARAMS,
        )(s0x, s1x, i2r, valr)

    return wrapped


def _make_pool():
    nrows = NP // EB
    per_w = 13

    def body(xin, belr, out, xbuf, belbuf, zbuf, accum, sem):
        c = lax.axis_index("c")
        s = lax.axis_index("s")

        @pl.when(s == 0)
        def _():
            def zb(r, cc):
                for p in range(4):
                    zbuf[r, pl.ds(p * 16, 16)] = jnp.zeros((16,), jnp.float32)
                return cc
            lax.fori_loop(0, NG, zb, 0)
            pltpu.sync_copy(zbuf, accum)

        plsc.subcore_barrier()
        w = c * NSUB + s

        def rowblk(k, carry):
            r = w * per_w + k

            @pl.when(r < nrows)
            def _():
                pltpu.sync_copy(xin.at[pl.ds(r * EB, EB)], xbuf)
                pltpu.sync_copy(belr.at[pl.ds(r, 1)], belbuf)
                pltpu.sync_copy(xbuf, accum.at[belbuf.at[0]], add=True)
            return carry

        lax.fori_loop(0, per_w, rowblk, 0)
        plsc.subcore_barrier()

        @pl.when(s == 0)
        def _():
            pltpu.sync_copy(accum, out.at[c])

    def wrapped(xin, belr):
        return pl.kernel(
            body,
            out_type=jax.ShapeDtypeStruct((2, NG, 64), jnp.float32),
            scratch_types=[
                pltpu.VMEM((EB, 64), jnp.float32),
                pltpu.VMEM((1, EB), jnp.int32),
                pltpu.VMEM((NG, 64), jnp.float32),
                pltpu.VMEM_SHARED((NG, 64), jnp.float32),
                pltpu.SemaphoreType.DMA,
            ], **_MESH, **_SC_PARAMS,
        )(xin, belr)

    return wrapped


_KERNELS = {}


def _get(kind, nrows=0):
    key = (kind, nrows)
    if key not in _KERNELS:
        if kind == "spmm":
            _KERNELS[key] = _spmm_builder(nrows, False)
        elif kind == "spmm_scaled":
            _KERNELS[key] = _spmm_builder(nrows, True)
        elif kind == "attn":
            _KERNELS[key] = _make_attn(nrows)
        else:
            _KERNELS[key] = _make_pool()
    return _KERNELS[key]


def _pad_rows(v, nrows, fill):
    n = v.shape[0]
    pad = nrows * EB - n
    return jnp.concatenate([v, jnp.full((pad,), fill, v.dtype)]).reshape(nrows, EB)


def _edge_rows(n):
    per = 2 * NSUB * CHUNK
    return ((n // EB + per - 1) // per) * per


def _quarters(y):
    y = jnp.pad(y, ((0, NP - N1), (0, 0)))
    return y.reshape(NP, 4, DQ).transpose(1, 0, 2)


def _expand16(v):
    return jnp.broadcast_to(v[:, None], (v.shape[0], DQ))


def _conv(x, idx, val, Ws, a, zeros2):
    d = Ws.shape[2]
    h = x @ Ws[0]
    s0x = _expand16(jnp.pad(h @ a[:d], (0, NP - N1)))
    s1x = _expand16(jnp.pad(h @ a[d:], (0, NP - N1)))
    nrows = _edge_rows(idx.shape[1])
    i2r = jnp.stack([_pad_rows(idx[0], nrows, N1),
                     _pad_rows(idx[1], nrows, N1)], axis=1)
    valr = _pad_rows(val, nrows, 0.0)
    wr, sspart = _get("attn", nrows)(s0x, s1x, i2r, valr)
    ssum = sspart[0, :, 0] + sspart[1, :, 0]
    dinv16 = _expand16(1.0 / (ssum + 1e-9))
    u2 = _get("spmm_scaled", nrows)(_quarters(x @ Ws[1]), i2r, wr,
                                    dinv16, _quarters(h))
    return _get("spmm_scaled", nrows)(u2, i2r, wr, dinv16, zeros2)


def kernel(x_0, x_1, x_2, up_idx, up_val, dn_idx, dn_val, inc_idx, inc_val, x_bel_1,
           W_in0, b_in0, W_in1, b_in1, W_in2, b_in2,
           W_up, a_up, W_dn, a_dn, W_har, W_out, b_out, W_ro, b_ro):
    x = x_1 @ W_in1 + b_in1
    zeros2 = jnp.zeros((4, NP, DQ), jnp.float32)
    z_up = _conv(x, up_idx, up_val, W_up, a_up, zeros2)
    z_dn = _conv(x, dn_idx, dn_val, W_dn, a_dn, zeros2)

    nb = _edge_rows(2 * up_idx.shape[1])
    bi2 = jnp.stack([_pad_rows(jnp.concatenate([up_idx[0], dn_idx[0]]), nb, N1),
                     _pad_rows(jnp.concatenate([up_idx[1], dn_idx[1]]), nb, N1)],
                    axis=1)
    bwx = _expand16(_pad_rows(jnp.concatenate([up_val, dn_val]) * (-EPS),
                              nb, 0.0).reshape(-1))
    y2 = _quarters(x @ W_har)
    for _ in range(ORDER):
        y2 = _get("spmm", nb)(y2, bi2, bwx, y2)

    z2 = jax.nn.relu(z_up + z_dn + y2)[:, :N1]
    x1_out = (z2[0] @ W_out[:DQ] + z2[1] @ W_out[DQ:2 * DQ]
              + z2[2] @ W_out[2 * DQ:3 * DQ] + z2[3] @ W_out[3 * DQ:] + b_out)
    xin = jnp.pad(x1_out, ((0, NP - N1), (0, 0)))
    belr = _pad_rows(x_bel_1, NP // EB, 0)
    parts = _get("pool")(xin, belr)
    pooled = parts[0] + parts[1]
    return pooled @ W_ro + b_ro


# speedup vs baseline: 1.0289x; 1.0289x over previous
import jax
import jax.numpy as jnp
from jax import lax
from jax.experimental import pallas as pl
from jax.experimental.pallas import tpu as pltpu
from jax.experimental.pallas import tpu_sc as plsc

N1 = 50000
NP = 50048
NG = 64
EPS = 0.1
ORDER = 5
DQ = 16
NSUB = 16
EB = 128
CHUNK = 8
CE = CHUNK * EB
RSLICE = NP // NSUB
_SC_PARAMS = dict(compiler_params=pltpu.CompilerParams(use_tc_tiling_on_sc=False))
_MESH = dict(mesh=plsc.VectorSubcoreMesh(core_axis_name="c", subcore_axis_name="s"))


def _wv16(buf, g):
    j = g // (EB // 16)
    m = pl.multiple_of((g % (EB // 16)) * 16, 16)
    return buf[j, pl.ds(m, 16)]


def _zero_rows(ref, n):
    def zb(i, cc):
        ref[i, :] = jnp.zeros((16,), jnp.float32)
        return cc
    lax.fori_loop(0, n, zb, 0, unroll=8)


_BLOCKS = ((0, CHUNK * EB), (1, CHUNK * EB), (2, RSLICE - 2 * CHUNK * EB))


def _init_accum_zero(zsrc, accum, off):
    for t, sz in _BLOCKS:
        pltpu.sync_copy(zsrc.at[pl.ds(0, sz)],
                        accum.at[pl.ds(off + t * CE, sz)])


def _spmm_builder(nrows, scaled):
    rows_per_tile = nrows // NSUB
    n_chunks = rows_per_tile // CHUNK
    assert n_chunks % 2 == 0

    def body(src, i2r, wr, dinv, base, out,
             ibA, ibB, wbA, wbB, gbA, gbB, accum, gsA, gsB, ssA, ssB):
        c = lax.axis_index("c")
        s = lax.axis_index("s")
        row0 = s * rows_per_tile
        off = s * RSLICE

        def fetch(k, ib, wb):
            r0 = row0 + k * CHUNK
            pltpu.sync_copy(i2r.at[pl.ds(r0, CHUNK)], ib)
            pltpu.sync_copy(wr.at[pl.ds(r0 * EB, CE)], wb)

        def escale(gb, wb):
            def eb(e, cc):
                gb[e, :] = gb[e, :] * wb[e, :]
                return cc
            lax.fori_loop(0, CE, eb, 0, unroll=8)

        for qi in range(2):
            q = 2 * c + qi

            def gdesc(ib, gb):
                return [pltpu.make_async_copy(src.at[q].at[ib.at[j, 1]],
                                              gb.at[pl.ds(j * EB, EB)],
                                              gsA if gb is gbA else gsB)
                        for j in range(CHUNK)]

            def sdesc(ib, gb):
                return [pltpu.make_async_copy(gb.at[pl.ds(j * EB, EB)],
                                              accum.at[ib.at[j, 0]],
                                              ssA if gb is gbA else ssB)
                        for j in range(CHUNK)]

            if scaled:
                _zero_rows(gbA, CE)
                _init_accum_zero(gbA, accum, off)
            else:
                pltpu.sync_copy(base.at[q].at[pl.ds(off, RSLICE)],
                                accum.at[pl.ds(off, RSLICE)])
            plsc.subcore_barrier()

            fetch(0, ibA, wbA)
            for d in gdesc(ibA, gbA):
                d.start()

            def half(k, ib, wb, gb, ibo, wbo, gbo):
                @pl.when(k >= 1)
                def _():
                    for d in sdesc(ibo, gbo):
                        d.wait()

                @pl.when(k + 1 < n_chunks)
                def _():
                    fetch(k + 1, ibo, wbo)
                    for d in gdesc(ibo, gbo):
                        d.start()

                for d in gdesc(ib, gb):
                    d.wait()
                escale(gb, wb)
                for d in sdesc(ib, gb):
                    d.start(add=True)

            def pair(t, carry):
                k0 = 2 * t
                half(k0, ibA, wbA, gbA, ibB, wbB, gbB)
                half(k0 + 1, ibB, wbB, gbB, ibA, wbA, gbA)
                return carry

            lax.fori_loop(0, n_chunks // 2, pair, 0)
            for d in sdesc(ibB, gbB):
                d.wait()
            plsc.subcore_barrier()
            if scaled:
                for t, sz in _BLOCKS:
                    rb = off + t * CE
                    pltpu.sync_copy(accum.at[pl.ds(rb, sz)], gbA.at[pl.ds(0, sz)])
                    pltpu.sync_copy(base.at[q].at[pl.ds(rb, sz)], gbB.at[pl.ds(0, sz)])
                    pltpu.sync_copy(dinv.at[pl.ds(rb, sz)], wbA.at[pl.ds(0, sz)])

                    def wfix(r, cc):
                        gbA[r, :] = gbA[r, :] * wbA[r, :] + gbB[r, :]
                        return cc

                    lax.fori_loop(0, sz, wfix, 0, unroll=8)
                    pltpu.sync_copy(gbA.at[pl.ds(0, sz)],
                                    out.at[q].at[pl.ds(rb, sz)])
            else:
                pltpu.sync_copy(accum.at[pl.ds(off, RSLICE)],
                                out.at[q].at[pl.ds(off, RSLICE)])
            plsc.subcore_barrier()

    scratch = [
        pltpu.VMEM((CHUNK, 2, EB), jnp.int32),
        pltpu.VMEM((CHUNK, 2, EB), jnp.int32),
        pltpu.VMEM((CE, DQ), jnp.float32),
        pltpu.VMEM((CE, DQ), jnp.float32),
        pltpu.VMEM((CE, DQ), jnp.float32),
        pltpu.VMEM((CE, DQ), jnp.float32),
        pltpu.VMEM_SHARED((NP, DQ), jnp.float32),
        pltpu.SemaphoreType.DMA,
        pltpu.SemaphoreType.DMA,
        pltpu.SemaphoreType.DMA,
        pltpu.SemaphoreType.DMA,
    ]
    out_t = jax.ShapeDtypeStruct((4, NP, DQ), jnp.float32)
    if scaled:
        def wrapped(src, i2r, wr, dinv, base):
            return pl.kernel(body, out_type=out_t, scratch_types=scratch,
                             **_MESH, **_SC_PARAMS)(src, i2r, wr, dinv, base)
    else:
        def wrapped(src, i2r, wr, base):
            def body2(src, i2r, wr, base, out, *rest):
                return body(src, i2r, wr, None, base, out, *rest)
            return pl.kernel(body2, out_type=out_t, scratch_types=scratch,
                             **_MESH, **_SC_PARAMS)(src, i2r, wr, base)
    return wrapped


def _make_attn(nrows):
    rows_per_core = nrows // 2
    rows_per_tile = rows_per_core // NSUB
    n_chunks = rows_per_tile // CHUNK

    def body(s0x, s1x, i2r, valr, wout, ssout,
             ibuf, vbuf, g0buf, g1buf, ssacc, sem):
        c = lax.axis_index("c")
        s = lax.axis_index("s")
        off = s * RSLICE
        _zero_rows(g0buf, CE)
        _init_accum_zero(g0buf, ssacc, off)
        plsc.subcore_barrier()
        row0 = c * rows_per_core + s * rows_per_tile

        def chunk_body(k, carry):
            r0 = row0 + k * CHUNK
            pltpu.sync_copy(i2r.at[pl.ds(r0, CHUNK)], ibuf)
            pltpu.sync_copy(valr.at[pl.ds(r0, CHUNK)], vbuf)
            cps = [pltpu.async_copy(s0x.at[ibuf.at[j, 0]],
                                    g0buf.at[pl.ds(j * EB, EB)], sem)
                   for j in range(CHUNK)]
            cps += [pltpu.async_copy(s1x.at[ibuf.at[j, 1]],
                                     g1buf.at[pl.ds(j * EB, EB)], sem)
                    for j in range(CHUNK)]
            for cp in cps:
                cp.wait()

            def egrp(g, carry2):
                vv = _wv16(vbuf, g)
                e0 = g * 16
                for l in range(16):
                    e = e0 + l
                    er = g0buf[e, :] + g1buf[e, :]
                    exr = jnp.exp(jnp.maximum(er, 0.2 * er))
                    g0buf[e, :] = exr
                    g1buf[e, :] = exr * vv[l]
                return carry2

            lax.fori_loop(0, CE // 16, egrp, 0)
            for j in range(CHUNK):
                pltpu.sync_copy(g0buf.at[pl.ds(j * EB, EB)],
                                ssacc.at[ibuf.at[j, 0]], add=True)
            pltpu.sync_copy(g1buf, wout.at[pl.ds(r0 * EB, CE)])
            return carry

        lax.fori_loop(0, n_chunks, chunk_body, 0)
        plsc.subcore_barrier()
        pltpu.sync_copy(ssacc.at[pl.ds(off, RSLICE)],
                        ssout.at[c].at[pl.ds(off, RSLICE)])

    def wrapped(s0x, s1x, i2r, valr):
        return pl.kernel(
            body,
            out_type=(jax.ShapeDtypeStruct((nrows * EB, DQ), jnp.float32),
                      jax.ShapeDtypeStruct((2, NP, DQ), jnp.float32)),
            scratch_types=[
                pltpu.VMEM((CHUNK, 2, EB), jnp.int32),
                pltpu.VMEM((CHUNK, EB), jnp.float32),
                pltpu.VMEM((CE, DQ), jnp.float32),
                pltpu.VMEM((CE, DQ), jnp.float32),
                pltpu.VMEM_SHARED((NP, DQ), jnp.float32),
                pltpu.SemaphoreType.DMA,
            ], **_MESH, **_SC_PARAMS,
        )(s0x, s1x, i2r, valr)

    return wrapped


def _make_pool():
    nrows = NP // EB
    per_w = 13

    def body(xin, belr, out, xbuf, belbuf, zbuf, accum, sem):
        c = lax.axis_index("c")
        s = lax.axis_index("s")

        @pl.when(s == 0)
        def _():
            def zb(r, cc):
                for p in range(4):
                    zbuf[r, pl.ds(p * 16, 16)] = jnp.zeros((16,), jnp.float32)
                return cc
            lax.fori_loop(0, NG, zb, 0)
            pltpu.sync_copy(zbuf, accum)

        plsc.subcore_barrier()
        w = c * NSUB + s

        def rowblk(k, carry):
            r = w * per_w + k

            @pl.when(r < nrows)
            def _():
                pltpu.sync_copy(xin.at[pl.ds(r * EB, EB)], xbuf)
                pltpu.sync_copy(belr.at[pl.ds(r, 1)], belbuf)
                pltpu.sync_copy(xbuf, accum.at[belbuf.at[0]], add=True)
            return carry

        lax.fori_loop(0, per_w, rowblk, 0)
        plsc.subcore_barrier()

        @pl.when(s == 0)
        def _():
            pltpu.sync_copy(accum, out.at[c])

    def wrapped(xin, belr):
        return pl.kernel(
            body,
            out_type=jax.ShapeDtypeStruct((2, NG, 64), jnp.float32),
            scratch_types=[
                pltpu.VMEM((EB, 64), jnp.float32),
                pltpu.VMEM((1, EB), jnp.int32),
                pltpu.VMEM((NG, 64), jnp.float32),
                pltpu.VMEM_SHARED((NG, 64), jnp.float32),
                pltpu.SemaphoreType.DMA,
            ], **_MESH, **_SC_PARAMS,
        )(xin, belr)

    return wrapped


_KERNELS = {}


def _get(kind, nrows=0):
    key = (kind, nrows)
    if key not in _KERNELS:
        if kind == "spmm":
            _KERNELS[key] = _spmm_builder(nrows, False)
        elif kind == "spmm_scaled":
            _KERNELS[key] = _spmm_builder(nrows, True)
        elif kind == "attn":
            _KERNELS[key] = _make_attn(nrows)
        else:
            _KERNELS[key] = _make_pool()
    return _KERNELS[key]


def _pad_rows(v, nrows, fill):
    n = v.shape[0]
    pad = nrows * EB - n
    return jnp.concatenate([v, jnp.full((pad,), fill, v.dtype)]).reshape(nrows, EB)


def _edge_rows(n):
    per = 2 * NSUB * CHUNK
    return ((n // EB + per - 1) // per) * per


def _quarters(y):
    y = jnp.pad(y, ((0, NP - N1), (0, 0)))
    return y.reshape(NP, 4, DQ).transpose(1, 0, 2)


def _expand16(v):
    return jnp.broadcast_to(v[:, None], (v.shape[0], DQ))


def _conv(x, idx, val, Ws, a, zeros2):
    d = Ws.shape[2]
    h = x @ Ws[0]
    s0x = _expand16(jnp.pad(h @ a[:d], (0, NP - N1)))
    s1x = _expand16(jnp.pad(h @ a[d:], (0, NP - N1)))
    nrows = _edge_rows(idx.shape[1])
    i2r = jnp.stack([_pad_rows(idx[0], nrows, N1),
                     _pad_rows(idx[1], nrows, N1)], axis=1)
    valr = _pad_rows(val, nrows, 0.0)
    wr, sspart = _get("attn", nrows)(s0x, s1x, i2r, valr)
    ssum = sspart[0, :, 0] + sspart[1, :, 0]
    dinv16 = _expand16(1.0 / (ssum + 1e-9))
    u2 = _get("spmm_scaled", nrows)(_quarters(x @ Ws[1]), i2r, wr,
                                    dinv16, _quarters(h))
    return _get("spmm_scaled", nrows)(u2, i2r, wr, dinv16, zeros2)


def kernel(x_0, x_1, x_2, up_idx, up_val, dn_idx, dn_val, inc_idx, inc_val, x_bel_1,
           W_in0, b_in0, W_in1, b_in1, W_in2, b_in2,
           W_up, a_up, W_dn, a_dn, W_har, W_out, b_out, W_ro, b_ro):
    x = x_1 @ W_in1 + b_in1
    zeros2 = jnp.zeros((4, NP, DQ), jnp.float32)
    z_up = _conv(x, up_idx, up_val, W_up, a_up, zeros2)
    z_dn = _conv(x, dn_idx, dn_val, W_dn, a_dn, zeros2)

    nb = _edge_rows(2 * up_idx.shape[1])
    bi2 = jnp.stack([_pad_rows(jnp.concatenate([up_idx[0], dn_idx[0]]), nb, N1),
                     _pad_rows(jnp.concatenate([up_idx[1], dn_idx[1]]), nb, N1)],
                    axis=1)
    bwx = _expand16(_pad_rows(jnp.concatenate([up_val, dn_val]) * (-EPS),
                              nb, 0.0).reshape(-1))
    y2 = _quarters(x @ W_har)
    for _ in range(ORDER):
        y2 = _get("spmm", nb)(y2, bi2, bwx, y2)

    z2 = jax.nn.relu(z_up + z_dn + y2)[:, :N1]
    x1_out = (z2[0] @ W_out[:DQ] + z2[1] @ W_out[DQ:2 * DQ]
              + z2[2] @ W_out[2 * DQ:3 * DQ] + z2[3] @ W_out[3 * DQ:] + b_out)
    xin = jnp.pad(x1_out, ((0, NP - N1), (0, 0)))
    belr = _pad_rows(x_bel_1, NP // EB, 0)
    parts = _get("pool")(xin, belr)
    pooled = parts[0] + parts[1]
    return pooled @ W_ro + b_ro
